# baseline (speedup 1.0000x reference)
import functools

import jax
import jax.numpy as jnp
from jax import lax
from jax.experimental import pallas as pl
from jax.experimental.pallas import tpu as pltpu
from jax.experimental.pallas import tpu_sc as plsc

HIDDEN = 64
VOCAB = 1000
NUM_EMB = 32
NC = 2
NS = 16
NW = NC * NS

TOKENS = 1024 * 50
B_PER_W = TOKENS // NW
CHUNK = 64
N_CHUNKS = B_PER_W // CHUNK


def _fused_mm_body(emb_ref, w_ref, out_ref):
    out_ref[...] = lax.dot_general(
        emb_ref[...], w_ref[...],
        dimension_numbers=(((1,), (1,)), ((), ())),
        preferred_element_type=jnp.float32,
    )


def _fused_table(emb, W):
    return pl.pallas_call(
        _fused_mm_body,
        out_shape=jax.ShapeDtypeStruct((NUM_EMB, VOCAB), jnp.float32),
    )(emb, W)


def _sc_gather_body(table_hbm, idx_hbm, out_hbm, idx_v, rows_v, sem):
    wid = lax.axis_index("s") * NC + lax.axis_index("c")
    base = wid * B_PER_W
    pltpu.sync_copy(idx_hbm.at[pl.ds(base, B_PER_W)], idx_v)

    def body(i, carry):
        off = i * CHUNK
        pltpu.async_copy(
            table_hbm.at[idx_v.at[pl.ds(off, CHUNK)]], rows_v, sem
        ).wait()
        pltpu.sync_copy(rows_v, out_hbm.at[pl.ds(base + off, CHUNK)])
        return carry

    lax.fori_loop(0, N_CHUNKS, body, 0)


_sc_gather = functools.partial(
    pl.kernel,
    out_type=jax.ShapeDtypeStruct((TOKENS, VOCAB), jnp.float32),
    mesh=plsc.VectorSubcoreMesh(core_axis_name="c", subcore_axis_name="s"),
    scratch_types=[
        pltpu.VMEM((B_PER_W,), jnp.int32),
        pltpu.VMEM((CHUNK, VOCAB), jnp.float32),
        pltpu.SemaphoreType.DMA,
    ],
    compiler_params=pltpu.CompilerParams(use_tc_tiling_on_sc=False),
)(_sc_gather_body)


def kernel(input_ids, emb_table, W):
    B, L = input_ids.shape
    fused = _fused_table(emb_table, W)
    ids_flat = input_ids.reshape(TOKENS).astype(jnp.int32)
    out = _sc_gather(fused, ids_flat)
    return out.reshape(B, L, VOCAB)


# speedup vs baseline: 1.4002x; 1.4002x over previous
import functools

import jax
import jax.numpy as jnp
from jax import lax
from jax.experimental import pallas as pl
from jax.experimental.pallas import tpu as pltpu
from jax.experimental.pallas import tpu_sc as plsc

HIDDEN = 64
VOCAB = 1000
NUM_EMB = 32
NC = 2
NS = 16
NW = NC * NS

TOKENS = 1024 * 50
B_PER_W = TOKENS // NW
CHUNK = 64
N_CHUNKS = B_PER_W // CHUNK


def _fused_mm_body(emb_ref, w_ref, out_ref):
    out_ref[...] = lax.dot_general(
        emb_ref[...], w_ref[...],
        dimension_numbers=(((1,), (1,)), ((), ())),
        preferred_element_type=jnp.float32,
    )


def _fused_table(emb, W):
    return pl.pallas_call(
        _fused_mm_body,
        out_shape=jax.ShapeDtypeStruct((NUM_EMB, VOCAB), jnp.float32),
    )(emb, W)


def _sc_gather_body(table_hbm, idx_hbm, out_hbm, table_s, idx_v, rows_v, sem):
    wid = lax.axis_index("s") * NC + lax.axis_index("c")
    base = wid * B_PER_W
    sid = lax.axis_index("s")

    @pl.when(sid == 0)
    def _():
        pltpu.sync_copy(table_hbm, table_s)

    plsc.subcore_barrier()
    pltpu.sync_copy(idx_hbm.at[pl.ds(base, B_PER_W)], idx_v)

    def body(i, carry):
        off = i * CHUNK
        pltpu.async_copy(
            table_s.at[idx_v.at[pl.ds(off, CHUNK)]], rows_v, sem
        ).wait()
        pltpu.sync_copy(rows_v, out_hbm.at[pl.ds(base + off, CHUNK)])
        return carry

    lax.fori_loop(0, N_CHUNKS, body, 0)


_sc_gather = functools.partial(
    pl.kernel,
    out_type=jax.ShapeDtypeStruct((TOKENS, VOCAB), jnp.float32),
    mesh=plsc.VectorSubcoreMesh(core_axis_name="c", subcore_axis_name="s"),
    scratch_types=[
        pltpu.VMEM_SHARED((NUM_EMB, VOCAB), jnp.float32),
        pltpu.VMEM((B_PER_W,), jnp.int32),
        pltpu.VMEM((CHUNK, VOCAB), jnp.float32),
        pltpu.SemaphoreType.DMA,
    ],
    compiler_params=pltpu.CompilerParams(use_tc_tiling_on_sc=False),
)(_sc_gather_body)


def kernel(input_ids, emb_table, W):
    B, L = input_ids.shape
    fused = _fused_table(emb_table, W)
    ids_flat = input_ids.reshape(TOKENS).astype(jnp.int32)
    out = _sc_gather(fused, ids_flat)
    return out.reshape(B, L, VOCAB)


# speedup vs baseline: 1.4955x; 1.0681x over previous
import functools

import jax
import jax.numpy as jnp
from jax import lax
from jax.experimental import pallas as pl
from jax.experimental.pallas import tpu as pltpu
from jax.experimental.pallas import tpu_sc as plsc

HIDDEN = 64
VOCAB = 1000
NUM_EMB = 32
NC = 2
NS = 16
NW = NC * NS

TOKENS = 1024 * 50
B_PER_W = TOKENS // NW
CHUNK = 40
N_CHUNKS = B_PER_W // CHUNK
N_PAIRS = N_CHUNKS // 2


def _fused_mm_body(emb_ref, w_ref, out_ref):
    out_ref[...] = lax.dot_general(
        emb_ref[...], w_ref[...],
        dimension_numbers=(((1,), (1,)), ((), ())),
        preferred_element_type=jnp.float32,
    )


def _fused_table(emb, W):
    return pl.pallas_call(
        _fused_mm_body,
        out_shape=jax.ShapeDtypeStruct((NUM_EMB, VOCAB), jnp.float32),
    )(emb, W)


def _sc_gather_body(
    table_hbm, idx_hbm, out_hbm, table_s, idx_v, rows_a, rows_b, gsem_a, gsem_b
):
    wid = lax.axis_index("s") * NC + lax.axis_index("c")
    base = wid * B_PER_W
    sid = lax.axis_index("s")

    @pl.when(sid == 0)
    def _():
        pltpu.sync_copy(table_hbm, table_s)

    plsc.subcore_barrier()
    pltpu.sync_copy(idx_hbm.at[pl.ds(base, B_PER_W)], idx_v)

    def gather_src(c):
        return table_s.at[idx_v.at[pl.ds(c * CHUNK, CHUNK)]]

    pltpu.async_copy(gather_src(0), rows_a, gsem_a)

    def body(j, carry):
        a = 2 * j
        b = a + 1
        pltpu.make_async_copy(gather_src(a), rows_a, gsem_a).wait()
        pltpu.async_copy(gather_src(b), rows_b, gsem_b)
        pltpu.sync_copy(rows_a, out_hbm.at[pl.ds(base + a * CHUNK, CHUNK)])
        pltpu.make_async_copy(gather_src(b), rows_b, gsem_b).wait()

        @pl.when(j < N_PAIRS - 1)
        def _():
            pltpu.async_copy(gather_src(a + 2), rows_a, gsem_a)

        pltpu.sync_copy(rows_b, out_hbm.at[pl.ds(base + b * CHUNK, CHUNK)])
        return carry

    lax.fori_loop(0, N_PAIRS, body, 0)


_sc_gather = functools.partial(
    pl.kernel,
    out_type=jax.ShapeDtypeStruct((TOKENS, VOCAB), jnp.float32),
    mesh=plsc.VectorSubcoreMesh(core_axis_name="c", subcore_axis_name="s"),
    scratch_types=[
        pltpu.VMEM_SHARED((NUM_EMB, VOCAB), jnp.float32),
        pltpu.VMEM((B_PER_W,), jnp.int32),
        pltpu.VMEM((CHUNK, VOCAB), jnp.float32),
        pltpu.VMEM((CHUNK, VOCAB), jnp.float32),
        pltpu.SemaphoreType.DMA,
        pltpu.SemaphoreType.DMA,
    ],
    compiler_params=pltpu.CompilerParams(use_tc_tiling_on_sc=False),
)(_sc_gather_body)


def kernel(input_ids, emb_table, W):
    B, L = input_ids.shape
    fused = _fused_table(emb_table, W)
    ids_flat = input_ids.reshape(TOKENS).astype(jnp.int32)
    out = _sc_gather(fused, ids_flat)
    return out.reshape(B, L, VOCAB)
